# baseline (speedup 1.0000x reference)
import functools

import jax
import jax.numpy as jnp
from jax import lax
from jax.experimental import pallas as pl
from jax.experimental.pallas import tpu as pltpu
from jax.experimental.pallas import tpu_sc as plsc

VOCAB = 1000000
EMBED_DIM = 32
BATCH = 4096
SEQ = 200

BB = 128
PLANE = BATCH * SEQ
LCH = 8
CH = LCH * BB
NCH = SEQ // LCH
UNROLL = 8


def _embed_body(xr_hbm, table_hbm, out_hbm, idx_v, rows0, rows1, tr0,
                gsem0, gsem1, osem0):
    info = plsc.get_sparse_core_info()
    nc = info.num_cores

    w = lax.axis_index("s") * nc + lax.axis_index("c")

    pltpu.sync_copy(xr_hbm.at[w], idx_v)

    rows = (rows0, rows1)
    gsem = (gsem0, gsem1)

    lane = lax.iota(jnp.int32, 16)
    c_lo = lane
    c_hi = lane + 16

    def start_gather(ch, k):
        pltpu.make_async_copy(
            table_hbm.at[idx_v.at[pl.ds(ch * CH, CH)]],
            rows[k], gsem[k]).start()

    def wait_gather(k):
        pltpu.make_async_copy(
            table_hbm.at[idx_v.at[pl.ds(0, CH)]],
            rows[k], gsem[k]).wait()

    def start_out(ch):
        pltpu.make_async_copy(
            tr0, out_hbm.at[:, pl.ds(ch * (32 * 1024) + w * 1024, CH)],
            osem0).start()

    def wait_out():
        pltpu.make_async_copy(
            tr0, out_hbm.at[:, pl.ds(0, CH)], osem0).wait()

    def transpose(k):
        rv = rows[k]

        def t_step(i, c2):
            for dj in range(UNROLL):
                r = i * UNROLL + dj
                r_vec = jnp.full((16,), r, jnp.int32)
                v0 = rv[r, pl.ds(0, 16)]
                v1 = rv[r, pl.ds(16, 16)]
                plsc.store_scatter(tr0, [c_lo, r_vec], v0)
                plsc.store_scatter(tr0, [c_hi, r_vec], v1)
            return c2

        lax.fori_loop(0, CH // UNROLL, t_step, 0)

    start_gather(0, 0)

    def pair(ci, carry):
        c0 = ci * 2

        start_gather(c0 + 1, 1)
        wait_gather(0)

        @pl.when(ci > 0)
        def _():
            wait_out()

        transpose(0)
        start_out(c0)

        @pl.when(c0 + 2 < NCH)
        def _():
            start_gather(c0 + 2, 0)

        wait_gather(1)
        wait_out()
        transpose(1)
        start_out(c0 + 1)
        return carry

    lax.fori_loop(0, NCH // 2, pair, 0)

    wait_gather(0)
    wait_out()
    transpose(0)
    start_out(NCH - 1)
    wait_out()


def kernel(x, table):
    mesh = plsc.VectorSubcoreMesh(core_axis_name="c", subcore_axis_name="s")

    f = functools.partial(
        pl.kernel,
        mesh=mesh,
        compiler_params=pltpu.CompilerParams(
            use_tc_tiling_on_sc=False, needs_layout_passes=False),
        out_type=jax.ShapeDtypeStruct((EMBED_DIM, PLANE), jnp.float32),
        scratch_types=[
            pltpu.VMEM((SEQ * BB,), jnp.int32),
            pltpu.VMEM((CH, EMBED_DIM), jnp.float32),
            pltpu.VMEM((CH, EMBED_DIM), jnp.float32),
            pltpu.VMEM((EMBED_DIM, CH), jnp.float32),
            pltpu.SemaphoreType.DMA,
            pltpu.SemaphoreType.DMA,
            pltpu.SemaphoreType.DMA,
        ],
    )(_embed_body)
    table4 = jnp.pad(table, ((0, 0), (0, 128 - EMBED_DIM))).reshape(
        4 * VOCAB, EMBED_DIM)
    xr = (jnp.transpose(x * 4).reshape(SEQ, BATCH // BB, BB)
          .transpose(1, 0, 2).reshape(BATCH // BB, SEQ * BB))
    out2 = f(xr, table4)
    v = out2.reshape(EMBED_DIM, SEQ // 8, BATCH // BB, 8, BB)
    return v.transpose(2, 4, 0, 1, 3).reshape(BATCH, EMBED_DIM, SEQ)


# speedup vs baseline: 1.0006x; 1.0006x over previous
import functools

import jax
import jax.numpy as jnp
from jax import lax
from jax.experimental import pallas as pl
from jax.experimental.pallas import tpu as pltpu
from jax.experimental.pallas import tpu_sc as plsc

VOCAB = 1000000
EMBED_DIM = 32
BATCH = 4096
SEQ = 200

BB = 128
PLANE = BATCH * SEQ
LCH = 8
CH = LCH * BB
NCH = SEQ // LCH
UNROLL = 8


def _embed_body(xr_hbm, table_hbm, out_hbm, idx_v, rows0, rows1, tr0,
                gsem0, gsem1, osem0):
    info = plsc.get_sparse_core_info()
    nc = info.num_cores

    w = lax.axis_index("s") * nc + lax.axis_index("c")

    pltpu.sync_copy(xr_hbm.at[w], idx_v)

    rows = (rows0, rows1)
    gsem = (gsem0, gsem1)

    lane = lax.iota(jnp.int32, 16)
    p_lo = lane * CH
    p_hi = (lane + 16) * CH

    def start_gather(ch, k):
        pltpu.make_async_copy(
            table_hbm.at[idx_v.at[pl.ds(ch * CH, CH)]],
            rows[k], gsem[k]).start()

    def wait_gather(k):
        pltpu.make_async_copy(
            table_hbm.at[idx_v.at[pl.ds(0, CH)]],
            rows[k], gsem[k]).wait()

    def start_out(ch):
        base = ch * (32 * 1024) + w * 1024
        for c in range(EMBED_DIM):
            pltpu.make_async_copy(
                tr0.at[pl.ds(c * CH, CH)],
                out_hbm.at[c, pl.ds(base, CH)], osem0).start()

    def wait_out():
        for c in range(EMBED_DIM):
            pltpu.make_async_copy(
                tr0.at[pl.ds(c * CH, CH)],
                out_hbm.at[c, pl.ds(0, CH)], osem0).wait()

    def transpose(k):
        rv = rows[k]

        def t_step(i, c2):
            for dj in range(UNROLL):
                r = i * UNROLL + dj
                v0 = rv[r, pl.ds(0, 16)]
                v1 = rv[r, pl.ds(16, 16)]
                plsc.store_scatter(tr0, [p_lo + r], v0)
                plsc.store_scatter(tr0, [p_hi + r], v1)
            return c2

        lax.fori_loop(0, CH // UNROLL, t_step, 0)

    start_gather(0, 0)

    def pair(ci, carry):
        c0 = ci * 2

        start_gather(c0 + 1, 1)
        wait_gather(0)

        @pl.when(ci > 0)
        def _():
            wait_out()

        transpose(0)
        start_out(c0)

        @pl.when(c0 + 2 < NCH)
        def _():
            start_gather(c0 + 2, 0)

        wait_gather(1)
        wait_out()
        transpose(1)
        start_out(c0 + 1)
        return carry

    lax.fori_loop(0, NCH // 2, pair, 0)

    wait_gather(0)
    wait_out()
    transpose(0)
    start_out(NCH - 1)
    wait_out()


def kernel(x, table):
    mesh = plsc.VectorSubcoreMesh(core_axis_name="c", subcore_axis_name="s")

    f = functools.partial(
        pl.kernel,
        mesh=mesh,
        compiler_params=pltpu.CompilerParams(
            use_tc_tiling_on_sc=False, needs_layout_passes=False),
        out_type=jax.ShapeDtypeStruct((EMBED_DIM, PLANE), jnp.float32),
        scratch_types=[
            pltpu.VMEM((SEQ * BB,), jnp.int32),
            pltpu.VMEM((CH, EMBED_DIM), jnp.float32),
            pltpu.VMEM((CH, EMBED_DIM), jnp.float32),
            pltpu.VMEM((EMBED_DIM * CH,), jnp.float32),
            pltpu.SemaphoreType.DMA,
            pltpu.SemaphoreType.DMA,
            pltpu.SemaphoreType.DMA,
        ],
    )(_embed_body)
    table4 = jnp.pad(table, ((0, 0), (0, 128 - EMBED_DIM))).reshape(
        4 * VOCAB, EMBED_DIM)
    xr = (jnp.transpose(x * 4).reshape(SEQ, BATCH // BB, BB)
          .transpose(1, 0, 2).reshape(BATCH // BB, SEQ * BB))
    out2 = f(xr, table4)
    v = out2.reshape(EMBED_DIM, SEQ // 8, BATCH // BB, 8, BB)
    return v.transpose(2, 4, 0, 1, 3).reshape(BATCH, EMBED_DIM, SEQ)


# speedup vs baseline: 1.5832x; 1.5822x over previous
import functools

import jax
import jax.numpy as jnp
from jax import lax
from jax.experimental import pallas as pl
from jax.experimental.pallas import tpu as pltpu
from jax.experimental.pallas import tpu_sc as plsc

VOCAB = 1000000
EMBED_DIM = 32
BATCH = 4096
SEQ = 200

BB = 128
PLANE = BATCH * SEQ
LCH = 8
CH = LCH * BB
NCH = SEQ // LCH
TRW = CH + 8
UNROLL = 8


def _embed_body(xr_hbm, table_hbm, out_hbm, idx_v, rows0, rows1, tr0,
                gsem0, gsem1, osem0):
    info = plsc.get_sparse_core_info()
    nc = info.num_cores

    w = lax.axis_index("s") * nc + lax.axis_index("c")

    pltpu.sync_copy(xr_hbm.at[w], idx_v)

    rows = (rows0, rows1)
    gsem = (gsem0, gsem1)

    lane = lax.iota(jnp.int32, 16)
    p_lo = lane * TRW
    p_hi = (lane + 16) * TRW

    def start_gather(ch, k):
        pltpu.make_async_copy(
            table_hbm.at[idx_v.at[pl.ds(ch * CH, CH)]],
            rows[k], gsem[k]).start()

    def wait_gather(k):
        pltpu.make_async_copy(
            table_hbm.at[idx_v.at[pl.ds(0, CH)]],
            rows[k], gsem[k]).wait()

    def start_out(ch):
        base = ch * (32 * 1024) + w * 1024
        for c in range(EMBED_DIM):
            pltpu.make_async_copy(
                tr0.at[pl.ds(c * TRW, CH)],
                out_hbm.at[c, pl.ds(base, CH)], osem0).start()

    def wait_out():
        for c in range(EMBED_DIM):
            pltpu.make_async_copy(
                tr0.at[pl.ds(c * TRW, CH)],
                out_hbm.at[c, pl.ds(0, CH)], osem0).wait()

    def transpose(k):
        rv = rows[k]

        def t_step(i, c2):
            for dj in range(UNROLL):
                r = i * UNROLL + dj
                v0 = rv[r, pl.ds(0, 16)]
                v1 = rv[r, pl.ds(16, 16)]
                plsc.store_scatter(tr0, [p_lo + r], v0)
                plsc.store_scatter(tr0, [p_hi + r], v1)
            return c2

        lax.fori_loop(0, CH // UNROLL, t_step, 0)

    start_gather(0, 0)

    def pair(ci, carry):
        c0 = ci * 2

        start_gather(c0 + 1, 1)
        wait_gather(0)

        @pl.when(ci > 0)
        def _():
            wait_out()

        transpose(0)
        start_out(c0)

        @pl.when(c0 + 2 < NCH)
        def _():
            start_gather(c0 + 2, 0)

        wait_gather(1)
        wait_out()
        transpose(1)
        start_out(c0 + 1)
        return carry

    lax.fori_loop(0, NCH // 2, pair, 0)

    wait_gather(0)
    wait_out()
    transpose(0)
    start_out(NCH - 1)
    wait_out()


def kernel(x, table):
    mesh = plsc.VectorSubcoreMesh(core_axis_name="c", subcore_axis_name="s")

    f = functools.partial(
        pl.kernel,
        mesh=mesh,
        compiler_params=pltpu.CompilerParams(
            use_tc_tiling_on_sc=False, needs_layout_passes=False),
        out_type=jax.ShapeDtypeStruct((EMBED_DIM, PLANE), jnp.float32),
        scratch_types=[
            pltpu.VMEM((SEQ * BB,), jnp.int32),
            pltpu.VMEM((CH, EMBED_DIM), jnp.float32),
            pltpu.VMEM((CH, EMBED_DIM), jnp.float32),
            pltpu.VMEM((EMBED_DIM * TRW,), jnp.float32),
            pltpu.SemaphoreType.DMA,
            pltpu.SemaphoreType.DMA,
            pltpu.SemaphoreType.DMA,
        ],
    )(_embed_body)
    table4 = jnp.pad(table, ((0, 0), (0, 128 - EMBED_DIM))).reshape(
        4 * VOCAB, EMBED_DIM)
    xr = (jnp.transpose(x * 4).reshape(SEQ, BATCH // BB, BB)
          .transpose(1, 0, 2).reshape(BATCH // BB, SEQ * BB))
    out2 = f(xr, table4)
    v = out2.reshape(EMBED_DIM, SEQ // 8, BATCH // BB, 8, BB)
    return v.transpose(2, 4, 0, 1, 3).reshape(BATCH, EMBED_DIM, SEQ)


# speedup vs baseline: 1.5936x; 1.0066x over previous
import functools

import jax
import jax.numpy as jnp
from jax import lax
from jax.experimental import pallas as pl
from jax.experimental.pallas import tpu as pltpu
from jax.experimental.pallas import tpu_sc as plsc

VOCAB = 1000000
EMBED_DIM = 32
BATCH = 4096
SEQ = 200

BB = 128
PLANE = BATCH * SEQ
LCH = 8
CH = LCH * BB
NCH = SEQ // LCH
TRW = CH + 8
UNROLL = 16


def _embed_body(xr_hbm, table_hbm, out_hbm, idx_v, rows0, rows1, tr0,
                gsem0, gsem1, osem0):
    info = plsc.get_sparse_core_info()
    nc = info.num_cores

    w = lax.axis_index("s") * nc + lax.axis_index("c")

    pltpu.sync_copy(xr_hbm.at[w], idx_v)

    rows = (rows0, rows1)
    gsem = (gsem0, gsem1)

    lane = lax.iota(jnp.int32, 16)
    p_lo = lane * TRW
    p_hi = (lane + 16) * TRW

    def start_gather(ch, k):
        pltpu.make_async_copy(
            table_hbm.at[idx_v.at[pl.ds(ch * CH, CH)]],
            rows[k], gsem[k]).start()

    def wait_gather(k):
        pltpu.make_async_copy(
            table_hbm.at[idx_v.at[pl.ds(0, CH)]],
            rows[k], gsem[k]).wait()

    def start_out(ch):
        base = ch * (32 * 1024) + w * 1024
        for c in range(EMBED_DIM):
            pltpu.make_async_copy(
                tr0.at[pl.ds(c * TRW, CH)],
                out_hbm.at[c, pl.ds(base, CH)], osem0).start()

    def wait_out():
        for c in range(EMBED_DIM):
            pltpu.make_async_copy(
                tr0.at[pl.ds(c * TRW, CH)],
                out_hbm.at[c, pl.ds(0, CH)], osem0).wait()

    def transpose(k):
        rv = rows[k]

        def t_step(i, c2):
            for dj in range(UNROLL):
                r = i * UNROLL + dj
                v0 = rv[r, pl.ds(0, 16)]
                v1 = rv[r, pl.ds(16, 16)]
                plsc.store_scatter(tr0, [p_lo + r], v0)
                plsc.store_scatter(tr0, [p_hi + r], v1)
            return c2

        lax.fori_loop(0, CH // UNROLL, t_step, 0)

    start_gather(0, 0)

    def pair(ci, carry):
        c0 = ci * 2

        start_gather(c0 + 1, 1)
        wait_gather(0)

        @pl.when(ci > 0)
        def _():
            wait_out()

        transpose(0)
        start_out(c0)

        @pl.when(c0 + 2 < NCH)
        def _():
            start_gather(c0 + 2, 0)

        wait_gather(1)
        wait_out()
        transpose(1)
        start_out(c0 + 1)
        return carry

    lax.fori_loop(0, NCH // 2, pair, 0)

    wait_gather(0)
    wait_out()
    transpose(0)
    start_out(NCH - 1)
    wait_out()


def kernel(x, table):
    mesh = plsc.VectorSubcoreMesh(core_axis_name="c", subcore_axis_name="s")

    f = functools.partial(
        pl.kernel,
        mesh=mesh,
        compiler_params=pltpu.CompilerParams(
            use_tc_tiling_on_sc=False, needs_layout_passes=False),
        out_type=jax.ShapeDtypeStruct((EMBED_DIM, PLANE), jnp.float32),
        scratch_types=[
            pltpu.VMEM((SEQ * BB,), jnp.int32),
            pltpu.VMEM((CH, EMBED_DIM), jnp.float32),
            pltpu.VMEM((CH, EMBED_DIM), jnp.float32),
            pltpu.VMEM((EMBED_DIM * TRW,), jnp.float32),
            pltpu.SemaphoreType.DMA,
            pltpu.SemaphoreType.DMA,
            pltpu.SemaphoreType.DMA,
        ],
    )(_embed_body)
    table4 = jnp.pad(table, ((0, 0), (0, 128 - EMBED_DIM))).reshape(
        4 * VOCAB, EMBED_DIM)
    xr = (jnp.transpose(x * 4).reshape(SEQ, BATCH // BB, BB)
          .transpose(1, 0, 2).reshape(BATCH // BB, SEQ * BB))
    out2 = f(xr, table4)
    v = out2.reshape(EMBED_DIM, SEQ // 8, BATCH // BB, 8, BB)
    return v.transpose(2, 4, 0, 1, 3).reshape(BATCH, EMBED_DIM, SEQ)


# speedup vs baseline: 1.6560x; 1.0392x over previous
import functools

import jax
import jax.numpy as jnp
from jax import lax
from jax.experimental import pallas as pl
from jax.experimental.pallas import tpu as pltpu
from jax.experimental.pallas import tpu_sc as plsc

VOCAB = 1000000
EMBED_DIM = 32
BATCH = 4096
SEQ = 200

BB = 128
PLANE = BATCH * SEQ
LCH = 4
CH = LCH * BB
NCH = SEQ // LCH
TRW = CH + 8
UNROLL = 16


def _embed_body(xr_hbm, table_hbm, out_hbm, idx_v, rows0, rows1, tr0, tr1,
                gsem0, gsem1, osem0, osem1):
    info = plsc.get_sparse_core_info()
    nc = info.num_cores

    w = lax.axis_index("s") * nc + lax.axis_index("c")

    pltpu.sync_copy(xr_hbm.at[w], idx_v)

    rows = (rows0, rows1)
    tr = (tr0, tr1)
    gsem = (gsem0, gsem1)
    osem = (osem0, osem1)

    lane = lax.iota(jnp.int32, 16)
    p_lo = lane * TRW
    p_hi = (lane + 16) * TRW

    def start_gather(ch, k):
        pltpu.make_async_copy(
            table_hbm.at[idx_v.at[pl.ds(ch * CH, CH)]],
            rows[k], gsem[k]).start()

    def wait_gather(k):
        pltpu.make_async_copy(
            table_hbm.at[idx_v.at[pl.ds(0, CH)]],
            rows[k], gsem[k]).wait()

    def start_out(ch, k):
        base = (ch // 2) * (32 * 1024) + w * 1024 + (ch % 2) * CH
        for c in range(EMBED_DIM):
            pltpu.make_async_copy(
                tr[k].at[pl.ds(c * TRW, CH)],
                out_hbm.at[c, pl.ds(base, CH)], osem[k]).start()

    def wait_out(k):
        for c in range(EMBED_DIM):
            pltpu.make_async_copy(
                tr[k].at[pl.ds(c * TRW, CH)],
                out_hbm.at[c, pl.ds(0, CH)], osem[k]).wait()

    def transpose(k):
        rv, tv = rows[k], tr[k]

        def t_step(i, c2):
            for dj in range(UNROLL):
                r = i * UNROLL + dj
                v0 = rv[r, pl.ds(0, 16)]
                v1 = rv[r, pl.ds(16, 16)]
                plsc.store_scatter(tv, [p_lo + r], v0)
                plsc.store_scatter(tv, [p_hi + r], v1)
            return c2

        lax.fori_loop(0, CH // UNROLL, t_step, 0)

    start_gather(0, 0)

    def pair(ci, carry):
        c0 = ci * 2

        start_gather(c0 + 1, 1)
        wait_gather(0)

        @pl.when(ci > 0)
        def _():
            wait_out(0)

        transpose(0)
        start_out(c0, 0)

        @pl.when(c0 + 2 < NCH)
        def _():
            start_gather(c0 + 2, 0)

        wait_gather(1)

        @pl.when(ci > 0)
        def _():
            wait_out(1)

        transpose(1)
        start_out(c0 + 1, 1)
        return carry

    lax.fori_loop(0, NCH // 2, pair, 0)
    wait_out(0)
    wait_out(1)


def kernel(x, table):
    mesh = plsc.VectorSubcoreMesh(core_axis_name="c", subcore_axis_name="s")

    f = functools.partial(
        pl.kernel,
        mesh=mesh,
        compiler_params=pltpu.CompilerParams(
            use_tc_tiling_on_sc=False, needs_layout_passes=False),
        out_type=jax.ShapeDtypeStruct((EMBED_DIM, PLANE), jnp.float32),
        scratch_types=[
            pltpu.VMEM((SEQ * BB,), jnp.int32),
            pltpu.VMEM((CH, EMBED_DIM), jnp.float32),
            pltpu.VMEM((CH, EMBED_DIM), jnp.float32),
            pltpu.VMEM((EMBED_DIM * TRW,), jnp.float32),
            pltpu.VMEM((EMBED_DIM * TRW,), jnp.float32),
            pltpu.SemaphoreType.DMA,
            pltpu.SemaphoreType.DMA,
            pltpu.SemaphoreType.DMA,
            pltpu.SemaphoreType.DMA,
        ],
    )(_embed_body)
    table4 = jnp.pad(table, ((0, 0), (0, 128 - EMBED_DIM))).reshape(
        4 * VOCAB, EMBED_DIM)
    xr = (jnp.transpose(x * 4).reshape(SEQ, BATCH // BB, BB)
          .transpose(1, 0, 2).reshape(BATCH // BB, SEQ * BB))
    out2 = f(xr, table4)
    v = out2.reshape(EMBED_DIM, SEQ // 8, BATCH // BB, 8, BB)
    return v.transpose(2, 4, 0, 1, 3).reshape(BATCH, EMBED_DIM, SEQ)
